# Initial kernel scaffold; baseline (speedup 1.0000x reference)
import functools
import math

import jax
import jax.numpy as jnp
import numpy as np
from jax import lax
from jax.experimental import pallas as pl
from jax.experimental.pallas import tpu as pltpu
from jax.experimental.pallas import tpu_sc as plsc

MIN_D = 0.01
MAX_D = 1000.0
N_POS = 64
N_HEADS = 16
SEQ = 2048
TOTAL = SEQ * SEQ

NC, NS, L = 2, 16, 16
NW = NC * NS
PER_W = TOTAL // NW
CHUNK = 4096
N_CHUNKS = PER_W // CHUNK
ROWS = CHUNK // 128

_LO = np.float32(math.log(float(np.float32(MIN_D))))
_HI = np.float32(math.log(float(np.float32(MAX_D))))
_HL = np.float32(_HI - _LO)
_HALF = np.float32(N_POS / 2.0)
_LN2 = np.float32(math.log(2.0))
_SQRT2 = np.float32(math.sqrt(2.0))


def _bucket_ids(x):
    neg = x < 0.0
    bits = plsc.bitcast(jnp.abs(x), jnp.int32)
    e = (bits >> 23) - 127
    m = plsc.bitcast((bits & 0x007FFFFF) | 0x3F800000, jnp.float32)
    big = m > _SQRT2
    m = jnp.where(big, m * np.float32(0.5), m)
    e = jnp.where(big, e + 1, e)
    s = (m - np.float32(1.0)) / (m + np.float32(1.0))
    z = s * s
    poly = np.float32(1.0) + z * (
        np.float32(1.0 / 3.0)
        + z * (np.float32(1.0 / 5.0) + z * np.float32(1.0 / 7.0))
    )
    ln_a = e.astype(jnp.float32) * _LN2 + np.float32(2.0) * s * poly
    u = jnp.minimum(jnp.maximum(ln_a, _LO), _HI)
    u = ((u - _LO) / _HL) * _HALF
    val = jnp.where(neg, u + (_HALF - np.float32(1.0)),
                    (_HALF - np.float32(1.0)) - u)
    i = val.astype(jnp.int32)
    return jnp.where(i < 0, i + (N_POS + 1), i)


def _body(d_hbm, table_hbm, out_hbm, d_v, idx_v, rows_v, sem):
    wid = lax.axis_index("s") * NC + lax.axis_index("c")
    base = wid * PER_W

    @pl.loop(0, N_CHUNKS)
    def _chunk(g):
        off = base + g * CHUNK
        pltpu.sync_copy(d_hbm.at[pl.ds(off, CHUNK)], d_v)

        @pl.loop(0, ROWS)
        def _compute(j):
            for k in range(128 // L):
                x = d_v[pl.ds(j * 128 + k * L, L)]
                idx_v[j, pl.ds(k * L, L)] = _bucket_ids(x)
            pltpu.async_copy(
                table_hbm.at[idx_v.at[j]],
                rows_v.at[pl.ds(j * 128, 128)],
                sem,
            )

        @pl.loop(0, ROWS)
        def _drain(j):
            pltpu.make_async_copy(
                table_hbm.at[idx_v.at[j]],
                rows_v.at[pl.ds(j * 128, 128)],
                sem,
            ).wait()

        pltpu.sync_copy(rows_v, out_hbm.at[pl.ds(off, CHUNK)])


@jax.jit
def _run(d_flat, table):
    mesh = plsc.VectorSubcoreMesh(core_axis_name="c", subcore_axis_name="s")
    return pl.kernel(
        _body,
        out_type=jax.ShapeDtypeStruct((TOTAL, N_HEADS), jnp.float32),
        mesh=mesh,
        scratch_types=[
            pltpu.VMEM((CHUNK,), jnp.float32),
            pltpu.VMEM((ROWS, 128), jnp.int32),
            pltpu.VMEM((CHUNK, N_HEADS), jnp.float32),
            pltpu.SemaphoreType.DMA,
        ],
    )(d_flat, table)


def kernel(d_mat, embeddings_table):
    out = _run(d_mat.reshape(TOTAL), embeddings_table)
    return out.reshape(SEQ, SEQ, N_HEADS)


# speedup vs baseline: 1.5299x; 1.5299x over previous
import functools
import math

import jax
import jax.numpy as jnp
import numpy as np
from jax import lax
from jax.experimental import pallas as pl
from jax.experimental.pallas import tpu as pltpu
from jax.experimental.pallas import tpu_sc as plsc

MIN_D = 0.01
MAX_D = 1000.0
N_POS = 64
N_HEADS = 16
SEQ = 2048
TOTAL = SEQ * SEQ

NC, NS, L = 2, 16, 16
NW = NC * NS
PER_W = TOTAL // NW
CHUNK = 4096
N_CHUNKS = PER_W // CHUNK
ROWS = CHUNK // 128

_LO = np.float32(math.log(float(np.float32(MIN_D))))
_HI = np.float32(math.log(float(np.float32(MAX_D))))
_HL = np.float32(_HI - _LO)
_HALF = np.float32(N_POS / 2.0)
_LN2 = np.float32(math.log(2.0))
_SQRT2 = np.float32(math.sqrt(2.0))


def _bucket_ids(x):
    neg = x < 0.0
    bits = lax.bitcast_convert_type(jnp.abs(x), jnp.int32)
    e = (bits >> 23) - 127
    m = lax.bitcast_convert_type((bits & 0x007FFFFF) | 0x3F800000, jnp.float32)
    big = m > _SQRT2
    m = jnp.where(big, m * np.float32(0.5), m)
    e = jnp.where(big, e + 1, e)
    s = (m - np.float32(1.0)) / (m + np.float32(1.0))
    z = s * s
    poly = np.float32(1.0) + z * (
        np.float32(1.0 / 3.0)
        + z * (np.float32(1.0 / 5.0) + z * np.float32(1.0 / 7.0))
    )
    ln_a = e.astype(jnp.float32) * _LN2 + np.float32(2.0) * s * poly
    u = jnp.minimum(jnp.maximum(ln_a, _LO), _HI)
    u = ((u - _LO) / _HL) * _HALF
    val = jnp.where(neg, u + (_HALF - np.float32(1.0)),
                    (_HALF - np.float32(1.0)) - u)
    i = val.astype(jnp.int32)
    return jnp.where(i < 0, i + (N_POS + 1), i)


def _body(d_hbm, table_hbm, out_hbm, d_v, idx_v, rows_v, sem):
    wid = lax.axis_index("s") * NC + lax.axis_index("c")
    base = wid * PER_W

    @pl.loop(0, N_CHUNKS)
    def _chunk(g):
        off = base + g * CHUNK
        pltpu.sync_copy(d_hbm.at[pl.ds(off, CHUNK)], d_v)

        @pl.loop(0, ROWS)
        def _compute(j):
            for k in range(128 // L):
                x = d_v[pl.ds(j * 128 + k * L, L)]
                idx_v[j, pl.ds(k * L, L)] = _bucket_ids(x)
            pltpu.async_copy(
                table_hbm.at[idx_v.at[j]],
                rows_v.at[pl.ds(j * 128, 128)],
                sem,
            )

        @pl.loop(0, ROWS)
        def _drain(j):
            pltpu.make_async_copy(
                table_hbm.at[idx_v.at[j]],
                rows_v.at[pl.ds(j * 128, 128)],
                sem,
            ).wait()

        pltpu.sync_copy(rows_v, out_hbm.at[pl.ds(off, CHUNK)])


@jax.jit
def _run(d_flat, table):
    mesh = plsc.VectorSubcoreMesh(core_axis_name="c", subcore_axis_name="s")
    return pl.kernel(
        _body,
        out_type=jax.ShapeDtypeStruct((TOTAL, N_HEADS), jnp.float32),
        mesh=mesh,
        scratch_types=[
            pltpu.VMEM((CHUNK,), jnp.float32),
            pltpu.VMEM((ROWS, 128), jnp.int32),
            pltpu.VMEM((CHUNK, N_HEADS), jnp.float32),
            pltpu.SemaphoreType.DMA,
        ],
        compiler_params=pltpu.CompilerParams(use_tc_tiling_on_sc=False),
    )(d_flat, table)


def kernel(d_mat, embeddings_table):
    out = _run(d_mat.reshape(TOTAL), embeddings_table)
    return out.reshape(SEQ, SEQ, N_HEADS)


# speedup vs baseline: 5.2632x; 3.4403x over previous
import math

import jax
import jax.numpy as jnp
import numpy as np
from jax import lax
from jax.experimental import pallas as pl
from jax.experimental.pallas import tpu as pltpu
from jax.experimental.pallas import tpu_sc as plsc

MIN_D = 0.01
MAX_D = 1000.0
N_POS = 64
N_HEADS = 16
SEQ = 2048
TOTAL = SEQ * SEQ

NC, NS, L = 2, 16, 16
NW = NC * NS
PER_W = TOTAL // NW
CHUNK = 2048
N_CHUNKS = PER_W // CHUNK
GROUPS = CHUNK // L
CWORDS = CHUNK * N_HEADS

_LO = np.float32(math.log(float(np.float32(MIN_D))))
_HI = np.float32(math.log(float(np.float32(MAX_D))))
_HL = np.float32(_HI - _LO)
_HALF = np.float32(N_POS / 2.0)
_LN2 = np.float32(math.log(2.0))
_SQRT2 = np.float32(math.sqrt(2.0))


def _bucket_ids(x):
    neg = x < 0.0
    bits = lax.bitcast_convert_type(jnp.abs(x), jnp.int32)
    e = (bits >> 23) - 127
    m = lax.bitcast_convert_type((bits & 0x007FFFFF) | 0x3F800000, jnp.float32)
    big = m > _SQRT2
    m = jnp.where(big, m * np.float32(0.5), m)
    e = jnp.where(big, e + 1, e)
    s = (m - np.float32(1.0)) / (m + np.float32(1.0))
    z = s * s
    poly = np.float32(1.0) + z * (
        np.float32(1.0 / 3.0)
        + z * (np.float32(1.0 / 5.0) + z * np.float32(1.0 / 7.0))
    )
    ln_a = e.astype(jnp.float32) * _LN2 + np.float32(2.0) * s * poly
    u = jnp.minimum(jnp.maximum(ln_a, _LO), _HI)
    u = ((u - _LO) / _HL) * _HALF
    val = jnp.where(neg, u + (_HALF - np.float32(1.0)),
                    (_HALF - np.float32(1.0)) - u)
    i = val.astype(jnp.int32)
    return jnp.where(i < 0, i + (N_POS + 1), i)


def _body(d_hbm, table_hbm, out_hbm, d_v, rows_v, table_v, sem0, sem1):
    wid = lax.axis_index("s") * NC + lax.axis_index("c")
    base = wid * PER_W
    sems = (sem0, sem1)

    pltpu.sync_copy(table_hbm, table_v)
    iota = lax.iota(jnp.int32, L)
    soff0 = iota * N_HEADS

    @pl.loop(0, N_CHUNKS, step=2)
    def _chunk(g0):
        for b in range(2):
            c = g0 + b
            @pl.when(c >= 2)
            def _():
                pltpu.make_async_copy(
                    rows_v.at[b],
                    out_hbm.at[pl.ds(0, CWORDS)],
                    sems[b],
                ).wait()

            off = base + c * CHUNK
            pltpu.sync_copy(d_hbm.at[pl.ds(off, CHUNK)], d_v.at[b])

            @pl.loop(0, GROUPS)
            def _group(v):
                x = d_v[b, pl.ds(v * L, L)]
                gidx = _bucket_ids(x) * N_HEADS
                soff = soff0 + v * (L * N_HEADS)
                for h in range(N_HEADS):
                    val = plsc.load_gather(table_v, [gidx + h])
                    plsc.store_scatter(rows_v.at[b], [soff + h], val)

            pltpu.async_copy(
                rows_v.at[b],
                out_hbm.at[pl.ds(off * N_HEADS, CWORDS)],
                sems[b],
            )

    for b in range(2):
        pltpu.make_async_copy(
            rows_v.at[b],
            out_hbm.at[pl.ds(0, CWORDS)],
            sems[b],
        ).wait()


@jax.jit
def _run(d_flat, table_flat):
    mesh = plsc.VectorSubcoreMesh(core_axis_name="c", subcore_axis_name="s")
    return pl.kernel(
        _body,
        out_type=jax.ShapeDtypeStruct((TOTAL * N_HEADS,), jnp.float32),
        mesh=mesh,
        scratch_types=[
            pltpu.VMEM((2, CHUNK), jnp.float32),
            pltpu.VMEM((2, CWORDS), jnp.float32),
            pltpu.VMEM(((N_POS + 1) * N_HEADS,), jnp.float32),
            pltpu.SemaphoreType.DMA,
            pltpu.SemaphoreType.DMA,
        ],
        compiler_params=pltpu.CompilerParams(
            use_tc_tiling_on_sc=False, needs_layout_passes=False
        ),
    )(d_flat, table_flat)


def kernel(d_mat, embeddings_table):
    out = _run(d_mat.reshape(TOTAL), embeddings_table.reshape(-1))
    return out.reshape(SEQ, SEQ, N_HEADS)


# speedup vs baseline: 5.2785x; 1.0029x over previous
import math

import jax
import jax.numpy as jnp
import numpy as np
from jax import lax
from jax.experimental import pallas as pl
from jax.experimental.pallas import tpu as pltpu
from jax.experimental.pallas import tpu_sc as plsc

MIN_D = 0.01
MAX_D = 1000.0
N_POS = 64
N_HEADS = 16
SEQ = 2048
TOTAL = SEQ * SEQ

NC, NS, L = 2, 16, 16
NW = NC * NS
PER_W = TOTAL // NW
CHUNK = 2048
N_CHUNKS = PER_W // CHUNK
GROUPS = CHUNK // L
CWORDS = CHUNK * N_HEADS

_LO = np.float32(math.log(float(np.float32(MIN_D))))
_HI = np.float32(math.log(float(np.float32(MAX_D))))
_HL = np.float32(_HI - _LO)
_HALF = np.float32(N_POS / 2.0)
_LN2 = np.float32(math.log(2.0))
_SQRT2 = np.float32(math.sqrt(2.0))


def _bucket_ids(x):
    neg = x < 0.0
    bits = lax.bitcast_convert_type(jnp.abs(x), jnp.int32)
    e = (bits >> 23) - 127
    m = lax.bitcast_convert_type((bits & 0x007FFFFF) | 0x3F800000, jnp.float32)
    big = m > _SQRT2
    m = jnp.where(big, m * np.float32(0.5), m)
    e = jnp.where(big, e + 1, e)
    s = (m - np.float32(1.0)) / (m + np.float32(1.0))
    z = s * s
    poly = np.float32(1.0) + z * (
        np.float32(1.0 / 3.0)
        + z * (np.float32(1.0 / 5.0) + z * np.float32(1.0 / 7.0))
    )
    ln_a = e.astype(jnp.float32) * _LN2 + np.float32(2.0) * s * poly
    u = jnp.minimum(jnp.maximum(ln_a, _LO), _HI)
    u = ((u - _LO) / _HL) * _HALF
    val = jnp.where(neg, u + (_HALF - np.float32(1.0)),
                    (_HALF - np.float32(1.0)) - u)
    i = val.astype(jnp.int32)
    return jnp.where(i < 0, i + (N_POS + 1), i)


def _body(d_hbm, table_hbm, out_hbm, d_v, rows_v, table_v, sem0, sem1):
    wid = lax.axis_index("s") * NC + lax.axis_index("c")
    base = wid * PER_W
    sems = (sem0, sem1)

    pltpu.sync_copy(table_hbm, table_v)
    iota = lax.iota(jnp.int32, L)
    soff0 = iota * N_HEADS

    @pl.loop(0, N_CHUNKS, step=2)
    def _chunk(g0):
        for b in range(2):
            c = g0 + b
            @pl.when(c >= 2)
            def _():
                pltpu.make_async_copy(
                    rows_v.at[b],
                    out_hbm.at[pl.ds(0, CWORDS)],
                    sems[b],
                ).wait()

            off = base + c * CHUNK
            pltpu.sync_copy(d_hbm.at[pl.ds(off, CHUNK)], d_v.at[b])

            @pl.loop(0, GROUPS, unroll=4)
            def _group(v):
                x = d_v[b, pl.ds(v * L, L)]
                gidx = _bucket_ids(x) * N_HEADS
                soff = soff0 + v * (L * N_HEADS)
                for h in range(N_HEADS):
                    val = plsc.load_gather(table_v, [gidx + h])
                    plsc.store_scatter(rows_v.at[b], [soff + h], val)

            pltpu.async_copy(
                rows_v.at[b],
                out_hbm.at[pl.ds(off * N_HEADS, CWORDS)],
                sems[b],
            )

    for b in range(2):
        pltpu.make_async_copy(
            rows_v.at[b],
            out_hbm.at[pl.ds(0, CWORDS)],
            sems[b],
        ).wait()


@jax.jit
def _run(d_flat, table_flat):
    mesh = plsc.VectorSubcoreMesh(core_axis_name="c", subcore_axis_name="s")
    return pl.kernel(
        _body,
        out_type=jax.ShapeDtypeStruct((TOTAL * N_HEADS,), jnp.float32),
        mesh=mesh,
        scratch_types=[
            pltpu.VMEM((2, CHUNK), jnp.float32),
            pltpu.VMEM((2, CWORDS), jnp.float32),
            pltpu.VMEM(((N_POS + 1) * N_HEADS,), jnp.float32),
            pltpu.SemaphoreType.DMA,
            pltpu.SemaphoreType.DMA,
        ],
        compiler_params=pltpu.CompilerParams(
            use_tc_tiling_on_sc=False, needs_layout_passes=False
        ),
    )(d_flat, table_flat)


def kernel(d_mat, embeddings_table):
    out = _run(d_mat.reshape(TOTAL), embeddings_table.reshape(-1))
    return out.reshape(SEQ, SEQ, N_HEADS)


# speedup vs baseline: 6.6620x; 1.2621x over previous
import math

import jax
import jax.numpy as jnp
import numpy as np
from jax import lax
from jax.experimental import pallas as pl
from jax.experimental.pallas import tpu as pltpu
from jax.experimental.pallas import tpu_sc as plsc

MIN_D = 0.01
MAX_D = 1000.0
N_POS = 64
N_HEADS = 16
SEQ = 2048
TOTAL = SEQ * SEQ

NC, NS, L = 2, 16, 16
NW = NC * NS
PER_W = TOTAL // NW
CHUNK = 2048
N_CHUNKS = PER_W // CHUNK
GROUPS = CHUNK // L
CWORDS = CHUNK * N_HEADS

_LO = np.float32(math.log(float(np.float32(MIN_D))))
_HI = np.float32(math.log(float(np.float32(MAX_D))))
_HL = np.float32(_HI - _LO)
_HALF = np.float32(N_POS / 2.0)
_LN2 = np.float32(math.log(2.0))
_SQRT2 = np.float32(math.sqrt(2.0))


def _bucket_ids(x):
    neg = x < 0.0
    bits = lax.bitcast_convert_type(jnp.abs(x), jnp.int32)
    e = (bits >> 23) - 127
    m = lax.bitcast_convert_type((bits & 0x007FFFFF) | 0x3F800000, jnp.float32)
    big = m > _SQRT2
    m = jnp.where(big, m * np.float32(0.5), m)
    e = jnp.where(big, e + 1, e)
    s = (m - np.float32(1.0)) / (m + np.float32(1.0))
    z = s * s
    poly = np.float32(1.0) + z * (
        np.float32(1.0 / 3.0)
        + z * (np.float32(1.0 / 5.0) + z * np.float32(1.0 / 7.0))
    )
    ln_a = e.astype(jnp.float32) * _LN2 + np.float32(2.0) * s * poly
    u = jnp.minimum(jnp.maximum(ln_a, _LO), _HI)
    u = ((u - _LO) / _HL) * _HALF
    val = jnp.where(neg, u + (_HALF - np.float32(1.0)),
                    (_HALF - np.float32(1.0)) - u)
    i = val.astype(jnp.int32)
    return jnp.where(i < 0, i + (N_POS + 1), i)


def _body(d_hbm, table_hbm, out_hbm, d_v, idx_v, rows_v, table_v, sem0, sem1):
    wid = lax.axis_index("s") * NC + lax.axis_index("c")
    base = wid * PER_W
    sems = (sem0, sem1)

    pltpu.sync_copy(table_hbm, table_v)
    iota = lax.iota(jnp.int32, L)

    @pl.loop(0, N_CHUNKS, step=2)
    def _chunk(g0):
        for b in range(2):
            c = g0 + b
            @pl.when(c >= 2)
            def _():
                pltpu.make_async_copy(
                    rows_v.at[b],
                    out_hbm.at[pl.ds(0, CWORDS)],
                    sems[b],
                ).wait()

            off = base + c * CHUNK
            pltpu.sync_copy(d_hbm.at[pl.ds(off, CHUNK)], d_v.at[b])

            @pl.loop(0, GROUPS, unroll=2)
            def _group(v):
                x = d_v[b, pl.ds(v * L, L)]
                gidx = _bucket_ids(x) * N_HEADS
                for u in range(L):
                    row = gidx[u] + iota
                    val = plsc.load_gather(table_v, [row])
                    rows_v[b, pl.ds((v * L + u) * N_HEADS, N_HEADS)] = val

            pltpu.async_copy(
                rows_v.at[b],
                out_hbm.at[pl.ds(off * N_HEADS, CWORDS)],
                sems[b],
            )

    for b in range(2):
        pltpu.make_async_copy(
            rows_v.at[b],
            out_hbm.at[pl.ds(0, CWORDS)],
            sems[b],
        ).wait()


@jax.jit
def _run(d_flat, table_flat):
    mesh = plsc.VectorSubcoreMesh(core_axis_name="c", subcore_axis_name="s")
    return pl.kernel(
        _body,
        out_type=jax.ShapeDtypeStruct((TOTAL * N_HEADS,), jnp.float32),
        mesh=mesh,
        scratch_types=[
            pltpu.VMEM((2, CHUNK), jnp.float32),
            pltpu.VMEM((2, CHUNK), jnp.int32),
            pltpu.VMEM((2, CWORDS), jnp.float32),
            pltpu.VMEM(((N_POS + 1) * N_HEADS,), jnp.float32),
            pltpu.SemaphoreType.DMA,
            pltpu.SemaphoreType.DMA,
        ],
        compiler_params=pltpu.CompilerParams(
            use_tc_tiling_on_sc=False, needs_layout_passes=False
        ),
    )(d_flat, table_flat)


def kernel(d_mat, embeddings_table):
    out = _run(d_mat.reshape(TOTAL), embeddings_table.reshape(-1))
    return out.reshape(SEQ, SEQ, N_HEADS)
